# baseline (speedup 1.0000x reference)
import jax
import jax.numpy as jnp
from jax.experimental import pallas as pl

L = 50000
NAA = 20
NP = 5
NT = 12
NCH = 3 + NT + 2
WCH = NP * NCH

CBLK = 8192


def _body(seqt_ref, bbt_ref, wallt_ref, out_ref):
    st = seqt_ref[...]
    bbt = bbt_ref[...]

    m = jnp.dot(wallt_ref[...], st, preferred_element_type=jnp.float32)

    eps = 1e-8
    n_at = bbt[0]
    ca = bbt[1]
    c_at = bbt[2]
    v1 = c_at - ca
    v2 = n_at - ca
    n1 = jnp.sqrt(jnp.sum(v1 * v1, axis=0, keepdims=True))
    e1 = v1 / (n1 + eps)
    d12 = jnp.sum(e1 * v2, axis=0, keepdims=True)
    u2 = v2 - e1 * d12
    n2 = jnp.sqrt(jnp.sum(u2 * u2, axis=0, keepdims=True))
    e2 = u2 / (n2 + eps)
    e3 = jnp.concatenate([
        e1[1:2] * e2[2:3] - e1[2:3] * e2[1:2],
        e1[2:3] * e2[0:1] - e1[0:1] * e2[2:3],
        e1[0:1] * e2[1:2] - e1[1:2] * e2[0:1],
    ], axis=0)

    gps = []
    for p in range(NP):
        b = p * NCH
        denom = m[b + 16:b + 17, :]
        inv = 1.0 / denom
        rot = (m[b:b + 1, :] * inv * e1
               + m[b + 1:b + 2, :] * inv * e2
               + m[b + 2:b + 3, :] * inv * e3 + ca)
        rest = m[b + 3:b + 16, :] * inv
        gps.append(jnp.concatenate([rot, rest, denom], axis=0))

    lane_j = jax.lax.broadcasted_iota(jnp.int32, (NCH, 128 * NP), 1)
    spread_idx = lane_j // NP
    lane_p = lane_j % NP
    masks = [(lane_p == p).astype(jnp.float32) for p in range(NP)]
    idx_cols = [spread_idx[:, k * 128:(k + 1) * 128] for k in range(NP)]
    mask_cols = [[masks[p][:, k * 128:(k + 1) * 128] for k in range(NP)]
                 for p in range(NP)]
    for u in range(CBLK // 128):
        srcs = [gps[p][:, u * 128:(u + 1) * 128] for p in range(NP)]
        for k in range(NP):
            acc = None
            for p in range(NP):
                sp = jnp.take_along_axis(srcs[p], idx_cols[k], axis=1)
                t = sp * mask_cols[p][k]
                acc = t if acc is None else acc + t
            base = (u * NP + k) * 128
            out_ref[:, base:base + 128] = acc


@jax.jit
def kernel(sequence_p1, bb_xyz_p1, sequence_p2, bb_xyz_p2,
           map_coords, map_types, map_weights, map_radii):
    cw = (map_coords * map_weights)[0]
    tw = (map_types * map_weights)[0]
    rw = map_radii[0] * map_weights[0]
    w = map_weights[0]
    wallt = jnp.concatenate([cw, tw, rw, w], axis=-1).reshape(NAA, WCH).T

    seqt = jnp.concatenate([sequence_p1.T, sequence_p2.T], axis=1)
    bbt = jnp.concatenate([jnp.transpose(bb_xyz_p1, (1, 2, 0)),
                           jnp.transpose(bb_xyz_p2, (1, 2, 0))], axis=2)

    nb = pl.cdiv(2 * L, CBLK)
    out = pl.pallas_call(
        _body,
        grid=(nb,),
        in_specs=[
            pl.BlockSpec((NAA, CBLK), lambda i: (0, i)),
            pl.BlockSpec((3, 3, CBLK), lambda i: (0, 0, i)),
            pl.BlockSpec((WCH, NAA), lambda i: (0, 0)),
        ],
        out_specs=pl.BlockSpec((NCH, CBLK * NP), lambda i: (0, i)),
        out_shape=jax.ShapeDtypeStruct((NCH, 2 * L * NP), jnp.float32),
    )(seqt, bbt, wallt)
    return out.T


# speedup vs baseline: 1.0045x; 1.0045x over previous
import jax
import jax.numpy as jnp
from jax.experimental import pallas as pl

L = 50000
NAA = 20
NP = 5
NT = 12
NCH = 3 + NT + 2
WCH = NP * NCH

CBLK = 4096


def _body(seqt_ref, bbt_ref, wallt_ref, out_ref):
    st = seqt_ref[...]
    bbt = bbt_ref[...]

    m = jnp.dot(wallt_ref[...], st, preferred_element_type=jnp.float32)

    eps = 1e-8
    n_at = bbt[0]
    ca = bbt[1]
    c_at = bbt[2]
    v1 = c_at - ca
    v2 = n_at - ca
    n1 = jnp.sqrt(jnp.sum(v1 * v1, axis=0, keepdims=True))
    e1 = v1 / (n1 + eps)
    d12 = jnp.sum(e1 * v2, axis=0, keepdims=True)
    u2 = v2 - e1 * d12
    n2 = jnp.sqrt(jnp.sum(u2 * u2, axis=0, keepdims=True))
    e2 = u2 / (n2 + eps)
    e3 = jnp.concatenate([
        e1[1:2] * e2[2:3] - e1[2:3] * e2[1:2],
        e1[2:3] * e2[0:1] - e1[0:1] * e2[2:3],
        e1[0:1] * e2[1:2] - e1[1:2] * e2[0:1],
    ], axis=0)

    gps = []
    for p in range(NP):
        b = p * NCH
        denom = m[b + 16:b + 17, :]
        inv = 1.0 / denom
        rot = (m[b:b + 1, :] * inv * e1
               + m[b + 1:b + 2, :] * inv * e2
               + m[b + 2:b + 3, :] * inv * e3 + ca)
        rest = m[b + 3:b + 16, :] * inv
        gps.append(jnp.concatenate([rot, rest, denom], axis=0))

    lane_j = jax.lax.broadcasted_iota(jnp.int32, (NCH, 128 * NP), 1)
    spread_idx = lane_j // NP
    lane_p = lane_j % NP
    masks = [(lane_p == p).astype(jnp.float32) for p in range(NP)]
    idx_cols = [spread_idx[:, k * 128:(k + 1) * 128] for k in range(NP)]
    mask_cols = [[masks[p][:, k * 128:(k + 1) * 128] for k in range(NP)]
                 for p in range(NP)]
    for u in range(CBLK // 128):
        srcs = [gps[p][:, u * 128:(u + 1) * 128] for p in range(NP)]
        for k in range(NP):
            acc = None
            for p in range(NP):
                sp = jnp.take_along_axis(srcs[p], idx_cols[k], axis=1)
                t = sp * mask_cols[p][k]
                acc = t if acc is None else acc + t
            base = (u * NP + k) * 128
            out_ref[:, base:base + 128] = acc


@jax.jit
def kernel(sequence_p1, bb_xyz_p1, sequence_p2, bb_xyz_p2,
           map_coords, map_types, map_weights, map_radii):
    cw = (map_coords * map_weights)[0]
    tw = (map_types * map_weights)[0]
    rw = map_radii[0] * map_weights[0]
    w = map_weights[0]
    wallt = jnp.concatenate([cw, tw, rw, w], axis=-1).reshape(NAA, WCH).T

    seqt = jnp.concatenate([sequence_p1.T, sequence_p2.T], axis=1)
    bbt = jnp.concatenate([jnp.transpose(bb_xyz_p1, (1, 2, 0)),
                           jnp.transpose(bb_xyz_p2, (1, 2, 0))], axis=2)

    nb = pl.cdiv(2 * L, CBLK)
    out = pl.pallas_call(
        _body,
        grid=(nb,),
        in_specs=[
            pl.BlockSpec((NAA, CBLK), lambda i: (0, i)),
            pl.BlockSpec((3, 3, CBLK), lambda i: (0, 0, i)),
            pl.BlockSpec((WCH, NAA), lambda i: (0, 0)),
        ],
        out_specs=pl.BlockSpec((NCH, CBLK * NP), lambda i: (0, i)),
        out_shape=jax.ShapeDtypeStruct((NCH, 2 * L * NP), jnp.float32),
    )(seqt, bbt, wallt)
    return out.T
